# baseline (speedup 1.0000x reference)
import functools

import jax
import jax.numpy as jnp
from jax import lax
from jax.experimental import pallas as pl
from jax.experimental.pallas import tpu as pltpu
from jax.experimental.pallas import tpu_sc as plsc

B, L = 8, 512
HID, NFD, EFD = 256, 64, 32
K, NL = 30, 3
CUTOFF = 20.0
NN = B * L
E = NN * K

F32 = jnp.float32


def _comps(v):
    return v[:, 0:1], v[:, 1:2], v[:, 2:3]


def _dihedral_sincos(p1, p2, p3, p4):
    v1 = p2 - p1
    v2 = p3 - p2
    v3 = p4 - p3
    v1x, v1y, v1z = _comps(v1)
    v2x, v2y, v2z = _comps(v2)
    v3x, v3y, v3z = _comps(v3)
    n1x = v1y * v2z - v1z * v2y
    n1y = v1z * v2x - v1x * v2z
    n1z = v1x * v2y - v1y * v2x
    n2x = v2y * v3z - v2z * v3y
    n2y = v2z * v3x - v2x * v3z
    n2z = v2x * v3y - v2y * v3x
    n1n = jnp.sqrt(n1x * n1x + n1y * n1y + n1z * n1z) + 1e-8
    n2n = jnp.sqrt(n2x * n2x + n2y * n2y + n2z * n2z) + 1e-8
    u1x, u1y, u1z = n1x / n1n, n1y / n1n, n1z / n1n
    u2x, u2y, u2z = n2x / n2n, n2y / n2n, n2z / n2n
    c = jnp.clip(u1x * u2x + u1y * u2y + u1z * u2z, -1.0, 1.0)
    sgn = jnp.sign(u1x * v3x + u1y * v3y + u1z * v3z)
    s = sgn * jnp.sqrt(jnp.maximum((1.0 - c) * (1.0 + c), 0.0))
    c = jnp.where(sgn == 0.0, 1.0, c)
    return s, c


def _ln(x, g, b):
    mu = jnp.mean(x, axis=-1, keepdims=True)
    var = jnp.mean((x - mu) ** 2, axis=-1, keepdims=True)
    return (x - mu) / jnp.sqrt(var + 1e-5) * g + b


def _dot(a, b):
    return jnp.dot(a, b, preferred_element_type=F32)


_GW = HID // 2


def _pack_bf16(y):
    ua = lax.bitcast_convert_type(y[:, :_GW], jnp.uint32)
    ub = lax.bitcast_convert_type(y[:, _GW:], jnp.uint32)
    ra = (ua + jnp.uint32(0x7FFF) + ((ua >> 16) & jnp.uint32(1))) >> 16
    rb = (ub + jnp.uint32(0x7FFF) + ((ub >> 16) & jnp.uint32(1))) >> 16
    return lax.bitcast_convert_type(ra | (rb << 16), F32)


def _unpack_bf16(w):
    u = lax.bitcast_convert_type(w, jnp.uint32)
    a = lax.bitcast_convert_type(u << 16, F32)
    b = lax.bitcast_convert_type(u & jnp.uint32(0xFFFF0000), F32)
    return jnp.concatenate([a, b], axis=1)


_BPP = 2


def _enc_knn_body(cprev_r, n_r, ca_r, c_r, nnext_r, nnom_r, canom_r, cat_r,
                  wd_r, bd_r, wn_r, bn_r, w1a_r, b1_r,
                  x_o, y_o, nbr_o, dist_o):
    for bb in range(_BPP):
        bidx = pl.program_id(0) * _BPP + bb
        n_ = n_r[bb]
        ca = ca_r[bb]
        c = c_r[bb]
        s1, c1 = _dihedral_sincos(cprev_r[bb], n_, ca, c)
        s2, c2 = _dihedral_sincos(n_, ca, c, nnext_r[bb])
        s3, c3 = _dihedral_sincos(ca, c, nnom_r[bb], canom_r[bb])
        enc = jnp.concatenate([s1, s2, s3, c1, c2, c3], axis=1)
        nf = _dot(enc, wd_r[...]) + bd_r[...]
        x = _dot(nf, wn_r[...]) + bn_r[...]
        x_o[bb * L:(bb + 1) * L, :] = x
        y_o[bb * L:(bb + 1) * L, :] = _pack_bf16(_dot(x, w1a_r[...]) + b1_r[...])

        cat = cat_r[bb]
        dx = ca[:, 0:1] - cat[0:1, :]
        dy = ca[:, 1:2] - cat[1:2, :]
        dz = ca[:, 2:3] - cat[2:3, :]
        ri = lax.broadcasted_iota(jnp.int32, (L, L), 0)
        ci = lax.broadcasted_iota(jnp.int32, (L, L), 1)
        d2 = dx * dx + dy * dy + dz * dz + jnp.where(ri == ci, F32(1e9), F32(0.0))
        work = -d2
        nbrs = []
        dists = []
        for _ in range(K):
            mx = jnp.max(work, axis=1, keepdims=True)
            sel = jnp.where(work == mx, ci, jnp.int32(1 << 30))
            idx = jnp.min(sel, axis=1, keepdims=True)
            nbrs.append(idx)
            dists.append(jnp.sqrt(jnp.maximum(-mx, 0.0) + 1e-12))
            work = jnp.where(ci == idx, F32(-3e38), work)
        nbr_o[bb] = jnp.concatenate(nbrs, axis=1) + bidx * L
        dist_o[bb] = jnp.concatenate(dists, axis=1)


_COORD_SPEC = pl.BlockSpec((_BPP, L, 3), lambda i: (i, 0, 0))


def _wspec(shape):
    return pl.BlockSpec(shape, lambda i: tuple(0 for _ in shape))


_enc_knn_call = pl.pallas_call(
    _enc_knn_body,
    grid=(B // _BPP,),
    in_specs=[_COORD_SPEC] * 7 + [pl.BlockSpec((_BPP, 3, L), lambda i: (i, 0, 0)),
                                  _wspec((6, NFD)), _wspec((1, NFD)),
                                  _wspec((NFD, HID)), _wspec((1, HID)),
                                  _wspec((HID, HID)), _wspec((1, HID))],
    out_specs=[pl.BlockSpec((_BPP * L, HID), lambda i: (i, 0)),
               pl.BlockSpec((_BPP * L, HID // 2), lambda i: (i, 0)),
               pl.BlockSpec((_BPP, L, K), lambda i: (i, 0, 0)),
               pl.BlockSpec((_BPP, L, K), lambda i: (i, 0, 0))],
    out_shape=[jax.ShapeDtypeStruct((NN, HID), F32),
               jax.ShapeDtypeStruct((NN, HID // 2), F32),
               jax.ShapeDtypeStruct((B, L, K), jnp.int32),
               jax.ShapeDtypeStruct((B, L, K), F32)],
)


_NW = 32
_EPW = E // _NW
_CH = 128
_NCH = _EPW // _CH


def _sc_gather_body(y_hbm, src_hbm, out_hbm, idx_v, rows_v, sem):
    wid = lax.axis_index("s") * 2 + lax.axis_index("c")
    base = pl.multiple_of(wid * _EPW, _CH)
    pltpu.sync_copy(src_hbm.at[pl.ds(base, _EPW)], idx_v)

    def chunk(j, carry):
        off = pl.multiple_of(base + j * _CH, _CH)
        pltpu.async_copy(y_hbm.at[idx_v.at[pl.ds(j * _CH, _CH)]], rows_v,
                         sem).wait()
        pltpu.sync_copy(rows_v, out_hbm.at[pl.ds(off, _CH)])
        return carry

    lax.fori_loop(0, _NCH, chunk, 0)


@functools.lru_cache(maxsize=1)
def _sc_gather_call():
    return pl.kernel(
        _sc_gather_body,
        out_type=jax.ShapeDtypeStruct((E, _GW), F32),
        mesh=plsc.VectorSubcoreMesh(core_axis_name="c", subcore_axis_name="s"),
        scratch_types=[pltpu.VMEM((_EPW,), jnp.int32),
                       pltpu.VMEM((_CH, _GW), F32),
                       pltpu.SemaphoreType.DMA],
    )


_NT = 128
_TE = _NT * K
_NTILES = NN // _NT


def _layer_body(g_r, d_r, x_r,
                w1b_r, mg1_r, mB1_r, mW2_r, mb2_r,
                uW1a_r, uW1b_r, ub1_r, ug1_r, uB1_r, uW2_r, ub2_r,
                w1an_r, b1n_r,
                x_o, y_o):
    dist = d_r[...]
    ii = lax.broadcasted_iota(jnp.int32, (1, EFD), 1).astype(F32)
    cen = ii * (CUTOFF / (EFD - 1))
    z = (dist - cen) * (EFD / CUTOFF)
    ea = jnp.exp(-0.5 * z * z)
    ep = _dot(ea, w1b_r[...])
    w = lax.bitcast_convert_type(g_r[...], jnp.uint32)
    ma = lax.bitcast_convert_type(w << 16, F32) + ep[:, :_GW]
    mb = lax.bitcast_convert_type(w & jnp.uint32(0xFFFF0000), F32) + ep[:, _GW:]
    mu = (jnp.sum(ma, axis=1, keepdims=True)
          + jnp.sum(mb, axis=1, keepdims=True)) * (1.0 / HID)
    q = (jnp.sum(ma * ma, axis=1, keepdims=True)
         + jnp.sum(mb * mb, axis=1, keepdims=True)) * (1.0 / HID)
    rinv = jax.lax.rsqrt(jnp.maximum(q - mu * mu, 0.0) + 1e-5)
    ka = (ma - mu) * rinv
    kb = (mb - mu) * rinv
    ka = jnp.maximum(ka * mg1_r[:, :_GW] + mB1_r[:, :_GW], 0.0)
    kb = jnp.maximum(kb * mg1_r[:, _GW:] + mB1_r[:, _GW:], 0.0)
    sa = jnp.sum(ka.reshape(_NT, K, _GW), axis=1)
    sb = jnp.sum(kb.reshape(_NT, K, _GW), axis=1)
    s = jnp.concatenate([sa, sb], axis=1)
    agg = _dot(s, mW2_r[...]) + F32(K) * mb2_r[...]
    u = _dot(x_r[...], uW1a_r[...]) + _dot(agg, uW1b_r[...]) + ub1_r[...]
    u = jnp.maximum(_ln(u, ug1_r[...], uB1_r[...]), 0.0)
    xn = _dot(u, uW2_r[...]) + ub2_r[...]
    x_o[...] = xn
    y_o[...] = _pack_bf16(_dot(xn, w1an_r[...]) + b1n_r[...])


_layer_call = pl.pallas_call(
    _layer_body,
    grid=(_NTILES,),
    in_specs=[pl.BlockSpec((_TE, HID // 2), lambda i: (i, 0)),
              pl.BlockSpec((_TE, 1), lambda i: (i, 0)),
              pl.BlockSpec((_NT, HID), lambda i: (i, 0)),
              _wspec((EFD, HID)), _wspec((1, HID)), _wspec((1, HID)),
              _wspec((HID, HID)), _wspec((1, HID)),
              _wspec((HID, HID)), _wspec((HID, HID)), _wspec((1, HID)),
              _wspec((1, HID)), _wspec((1, HID)), _wspec((HID, HID)),
              _wspec((1, HID)),
              _wspec((HID, HID)), _wspec((1, HID))],
    out_specs=[pl.BlockSpec((_NT, HID), lambda i: (i, 0)),
               pl.BlockSpec((_NT, HID // 2), lambda i: (i, 0))],
    out_shape=[jax.ShapeDtypeStruct((NN, HID), F32),
               jax.ShapeDtypeStruct((NN, HID // 2), F32)],
)


def _row(v):
    return v.reshape(1, -1)


def kernel(backbone_coords, params):
    coords = backbone_coords
    n_ = coords[:, :, 0, :]
    ca = coords[:, :, 1, :]
    c = coords[:, :, 2, :]
    cprev = jnp.concatenate([c[:, :1], c[:, :-1]], axis=1)
    nnext = jnp.concatenate([n_[:, 1:], n_[:, -1:]], axis=1)
    nnom = jnp.concatenate([n_[:, 1:], ca[:, -1:]], axis=1)
    canom = jnp.concatenate([ca[:, 1:], ca[:, -1:]], axis=1)
    cat = ca.transpose(0, 2, 1)

    lp0 = params["layers"][0]
    x, y, nbrg, dist = _enc_knn_call(
        cprev, n_, ca, c, nnext, nnom, canom, cat,
        params["dproj"]["W"], _row(params["dproj"]["b"]),
        params["nproj"]["W"], _row(params["nproj"]["b"]),
        lp0["mW1"][:HID], _row(lp0["mb1"]))

    src = nbrg.reshape(E)
    dist_e = dist.reshape(E, 1)
    zW = jnp.zeros((HID, HID), F32)
    zb = jnp.zeros((1, HID), F32)

    for l in range(NL):
        lp = params["layers"][l]
        if l + 1 < NL:
            lpn = params["layers"][l + 1]
            w1an, b1n = lpn["mW1"][:HID], _row(lpn["mb1"])
        else:
            w1an, b1n = zW, zb
        g = _sc_gather_call()(y, src)
        x, y = _layer_call(
            g, dist_e, x,
            lp["mW1"][HID:], _row(lp["mg1"]), _row(lp["mB1"]),
            lp["mW2"], _row(lp["mb2"]),
            lp["uW1"][:HID], lp["uW1"][HID:], _row(lp["ub1"]),
            _row(lp["ug1"]), _row(lp["uB1"]), lp["uW2"], _row(lp["ub2"]),
            w1an, b1n)

    return x.reshape(B, L, HID)


# speedup vs baseline: 1.0644x; 1.0644x over previous
import functools

import jax
import jax.numpy as jnp
from jax import lax
from jax.experimental import pallas as pl
from jax.experimental.pallas import tpu as pltpu
from jax.experimental.pallas import tpu_sc as plsc

B, L = 8, 512
HID, NFD, EFD = 256, 64, 32
K, NL = 30, 3
CUTOFF = 20.0
NN = B * L
E = NN * K

F32 = jnp.float32


def _comps(v):
    return v[:, 0:1], v[:, 1:2], v[:, 2:3]


def _dihedral_sincos(p1, p2, p3, p4):
    v1 = p2 - p1
    v2 = p3 - p2
    v3 = p4 - p3
    v1x, v1y, v1z = _comps(v1)
    v2x, v2y, v2z = _comps(v2)
    v3x, v3y, v3z = _comps(v3)
    n1x = v1y * v2z - v1z * v2y
    n1y = v1z * v2x - v1x * v2z
    n1z = v1x * v2y - v1y * v2x
    n2x = v2y * v3z - v2z * v3y
    n2y = v2z * v3x - v2x * v3z
    n2z = v2x * v3y - v2y * v3x
    n1n = jnp.sqrt(n1x * n1x + n1y * n1y + n1z * n1z) + 1e-8
    n2n = jnp.sqrt(n2x * n2x + n2y * n2y + n2z * n2z) + 1e-8
    u1x, u1y, u1z = n1x / n1n, n1y / n1n, n1z / n1n
    u2x, u2y, u2z = n2x / n2n, n2y / n2n, n2z / n2n
    c = jnp.clip(u1x * u2x + u1y * u2y + u1z * u2z, -1.0, 1.0)
    sgn = jnp.sign(u1x * v3x + u1y * v3y + u1z * v3z)
    s = sgn * jnp.sqrt(jnp.maximum((1.0 - c) * (1.0 + c), 0.0))
    c = jnp.where(sgn == 0.0, 1.0, c)
    return s, c


def _ln(x, g, b):
    mu = jnp.mean(x, axis=-1, keepdims=True)
    var = jnp.mean((x - mu) ** 2, axis=-1, keepdims=True)
    return (x - mu) / jnp.sqrt(var + 1e-5) * g + b


def _dot(a, b):
    return jnp.dot(a, b, preferred_element_type=F32)


_GW = HID // 2


def _pack_bf16(y):
    ua = lax.bitcast_convert_type(y[:, :_GW], jnp.uint32)
    ub = lax.bitcast_convert_type(y[:, _GW:], jnp.uint32)
    ra = (ua + jnp.uint32(0x7FFF) + ((ua >> 16) & jnp.uint32(1))) >> 16
    rb = (ub + jnp.uint32(0x7FFF) + ((ub >> 16) & jnp.uint32(1))) >> 16
    return lax.bitcast_convert_type(ra | (rb << 16), F32)


def _unpack_bf16(w):
    u = lax.bitcast_convert_type(w, jnp.uint32)
    a = lax.bitcast_convert_type(u << 16, F32)
    b = lax.bitcast_convert_type(u & jnp.uint32(0xFFFF0000), F32)
    return jnp.concatenate([a, b], axis=1)


_BPP = 1


def _enc_knn_body(cprev_r, n_r, ca_r, c_r, nnext_r, nnom_r, canom_r, cat_r,
                  wd_r, bd_r, wn_r, bn_r, w1a_r, b1_r,
                  x_o, y_o, nbr_o, dist_o):
    for bb in range(_BPP):
        bidx = pl.program_id(0) * _BPP + bb
        n_ = n_r[bb]
        ca = ca_r[bb]
        c = c_r[bb]
        s1, c1 = _dihedral_sincos(cprev_r[bb], n_, ca, c)
        s2, c2 = _dihedral_sincos(n_, ca, c, nnext_r[bb])
        s3, c3 = _dihedral_sincos(ca, c, nnom_r[bb], canom_r[bb])
        enc = jnp.concatenate([s1, s2, s3, c1, c2, c3], axis=1)
        nf = _dot(enc, wd_r[...]) + bd_r[...]
        x = _dot(nf, wn_r[...]) + bn_r[...]
        x_o[bb * L:(bb + 1) * L, :] = x
        y_o[bb * L:(bb + 1) * L, :] = _pack_bf16(_dot(x, w1a_r[...]) + b1_r[...])

        cat = cat_r[bb]
        dx = ca[:, 0:1] - cat[0:1, :]
        dy = ca[:, 1:2] - cat[1:2, :]
        dz = ca[:, 2:3] - cat[2:3, :]
        ri = lax.broadcasted_iota(jnp.int32, (L, L), 0)
        ci = lax.broadcasted_iota(jnp.int32, (L, L), 1)
        d2 = dx * dx + dy * dy + dz * dz + jnp.where(ri == ci, F32(1e9), F32(0.0))
        work = -d2
        nbrs = []
        dists = []
        for _ in range(K):
            mx = jnp.max(work, axis=1, keepdims=True)
            sel = jnp.where(work == mx, ci, jnp.int32(1 << 30))
            idx = jnp.min(sel, axis=1, keepdims=True)
            nbrs.append(idx)
            dists.append(jnp.sqrt(jnp.maximum(-mx, 0.0) + 1e-12))
            work = jnp.where(ci == idx, F32(-3e38), work)
        nbr_o[bb] = jnp.concatenate(nbrs, axis=1) + bidx * L
        dist_o[bb] = jnp.concatenate(dists, axis=1)


_COORD_SPEC = pl.BlockSpec((_BPP, L, 3), lambda i: (i, 0, 0))


def _wspec(shape):
    return pl.BlockSpec(shape, lambda i: tuple(0 for _ in shape))


_enc_knn_call = pl.pallas_call(
    _enc_knn_body,
    grid=(B // _BPP,),
    in_specs=[_COORD_SPEC] * 7 + [pl.BlockSpec((_BPP, 3, L), lambda i: (i, 0, 0)),
                                  _wspec((6, NFD)), _wspec((1, NFD)),
                                  _wspec((NFD, HID)), _wspec((1, HID)),
                                  _wspec((HID, HID)), _wspec((1, HID))],
    out_specs=[pl.BlockSpec((_BPP * L, HID), lambda i: (i, 0)),
               pl.BlockSpec((_BPP * L, HID // 2), lambda i: (i, 0)),
               pl.BlockSpec((_BPP, L, K), lambda i: (i, 0, 0)),
               pl.BlockSpec((_BPP, L, K), lambda i: (i, 0, 0))],
    out_shape=[jax.ShapeDtypeStruct((NN, HID), F32),
               jax.ShapeDtypeStruct((NN, HID // 2), F32),
               jax.ShapeDtypeStruct((B, L, K), jnp.int32),
               jax.ShapeDtypeStruct((B, L, K), F32)],
)


_NW = 32
_EPW = E // _NW
_CH = 128
_NCH = _EPW // _CH


def _sc_gather_body(y_hbm, src_hbm, out_hbm, idx_v, rows_v, sem):
    wid = lax.axis_index("s") * 2 + lax.axis_index("c")
    base = pl.multiple_of(wid * _EPW, _CH)
    pltpu.sync_copy(src_hbm.at[pl.ds(base, _EPW)], idx_v)

    def chunk(j, carry):
        off = pl.multiple_of(base + j * _CH, _CH)
        pltpu.async_copy(y_hbm.at[idx_v.at[pl.ds(j * _CH, _CH)]], rows_v,
                         sem).wait()
        pltpu.sync_copy(rows_v, out_hbm.at[pl.ds(off, _CH)])
        return carry

    lax.fori_loop(0, _NCH, chunk, 0)


@functools.lru_cache(maxsize=1)
def _sc_gather_call():
    return pl.kernel(
        _sc_gather_body,
        out_type=jax.ShapeDtypeStruct((E, _GW), F32),
        mesh=plsc.VectorSubcoreMesh(core_axis_name="c", subcore_axis_name="s"),
        scratch_types=[pltpu.VMEM((_EPW,), jnp.int32),
                       pltpu.VMEM((_CH, _GW), F32),
                       pltpu.SemaphoreType.DMA],
    )


_NT = 128
_TE = _NT * K
_NTILES = NN // _NT


def _layer_body(g_r, d_r, x_r,
                w1b_r, mg1_r, mB1_r, mW2_r, mb2_r,
                uW1a_r, uW1b_r, ub1_r, ug1_r, uB1_r, uW2_r, ub2_r,
                w1an_r, b1n_r,
                x_o, y_o):
    dist = d_r[...]
    ii = lax.broadcasted_iota(jnp.int32, (1, EFD), 1).astype(F32)
    cen = ii * (CUTOFF / (EFD - 1))
    z = (dist - cen) * (EFD / CUTOFF)
    ea = jnp.exp(-0.5 * z * z)
    ep = _dot(ea, w1b_r[...])
    w = lax.bitcast_convert_type(g_r[...], jnp.uint32)
    ma = lax.bitcast_convert_type(w << 16, F32) + ep[:, :_GW]
    mb = lax.bitcast_convert_type(w & jnp.uint32(0xFFFF0000), F32) + ep[:, _GW:]
    mu = (jnp.sum(ma, axis=1, keepdims=True)
          + jnp.sum(mb, axis=1, keepdims=True)) * (1.0 / HID)
    q = (jnp.sum(ma * ma, axis=1, keepdims=True)
         + jnp.sum(mb * mb, axis=1, keepdims=True)) * (1.0 / HID)
    rinv = jax.lax.rsqrt(jnp.maximum(q - mu * mu, 0.0) + 1e-5)
    ka = (ma - mu) * rinv
    kb = (mb - mu) * rinv
    ka = jnp.maximum(ka * mg1_r[:, :_GW] + mB1_r[:, :_GW], 0.0)
    kb = jnp.maximum(kb * mg1_r[:, _GW:] + mB1_r[:, _GW:], 0.0)
    sa = jnp.sum(ka.reshape(_NT, K, _GW), axis=1)
    sb = jnp.sum(kb.reshape(_NT, K, _GW), axis=1)
    s = jnp.concatenate([sa, sb], axis=1)
    agg = _dot(s, mW2_r[...]) + F32(K) * mb2_r[...]
    u = _dot(x_r[...], uW1a_r[...]) + _dot(agg, uW1b_r[...]) + ub1_r[...]
    u = jnp.maximum(_ln(u, ug1_r[...], uB1_r[...]), 0.0)
    xn = _dot(u, uW2_r[...]) + ub2_r[...]
    x_o[...] = xn
    y_o[...] = _pack_bf16(_dot(xn, w1an_r[...]) + b1n_r[...])


_layer_call = pl.pallas_call(
    _layer_body,
    grid=(_NTILES,),
    in_specs=[pl.BlockSpec((_TE, HID // 2), lambda i: (i, 0)),
              pl.BlockSpec((_TE, 1), lambda i: (i, 0)),
              pl.BlockSpec((_NT, HID), lambda i: (i, 0)),
              _wspec((EFD, HID)), _wspec((1, HID)), _wspec((1, HID)),
              _wspec((HID, HID)), _wspec((1, HID)),
              _wspec((HID, HID)), _wspec((HID, HID)), _wspec((1, HID)),
              _wspec((1, HID)), _wspec((1, HID)), _wspec((HID, HID)),
              _wspec((1, HID)),
              _wspec((HID, HID)), _wspec((1, HID))],
    out_specs=[pl.BlockSpec((_NT, HID), lambda i: (i, 0)),
               pl.BlockSpec((_NT, HID // 2), lambda i: (i, 0))],
    out_shape=[jax.ShapeDtypeStruct((NN, HID), F32),
               jax.ShapeDtypeStruct((NN, HID // 2), F32)],
)


def _row(v):
    return v.reshape(1, -1)


def kernel(backbone_coords, params):
    coords = backbone_coords
    n_ = coords[:, :, 0, :]
    ca = coords[:, :, 1, :]
    c = coords[:, :, 2, :]
    cprev = jnp.concatenate([c[:, :1], c[:, :-1]], axis=1)
    nnext = jnp.concatenate([n_[:, 1:], n_[:, -1:]], axis=1)
    nnom = jnp.concatenate([n_[:, 1:], ca[:, -1:]], axis=1)
    canom = jnp.concatenate([ca[:, 1:], ca[:, -1:]], axis=1)
    cat = ca.transpose(0, 2, 1)

    lp0 = params["layers"][0]
    x, y, nbrg, dist = _enc_knn_call(
        cprev, n_, ca, c, nnext, nnom, canom, cat,
        params["dproj"]["W"], _row(params["dproj"]["b"]),
        params["nproj"]["W"], _row(params["nproj"]["b"]),
        lp0["mW1"][:HID], _row(lp0["mb1"]))

    src = nbrg.reshape(E)
    dist_e = dist.reshape(E, 1)
    zW = jnp.zeros((HID, HID), F32)
    zb = jnp.zeros((1, HID), F32)

    for l in range(NL):
        lp = params["layers"][l]
        if l + 1 < NL:
            lpn = params["layers"][l + 1]
            w1an, b1n = lpn["mW1"][:HID], _row(lpn["mb1"])
        else:
            w1an, b1n = zW, zb
        g = _sc_gather_call()(y, src)
        x, y = _layer_call(
            g, dist_e, x,
            lp["mW1"][HID:], _row(lp["mg1"]), _row(lp["mB1"]),
            lp["mW2"], _row(lp["mb2"]),
            lp["uW1"][:HID], lp["uW1"][HID:], _row(lp["ub1"]),
            _row(lp["ug1"]), _row(lp["uB1"]), lp["uW2"], _row(lp["ub2"]),
            w1an, b1n)

    return x.reshape(B, L, HID)


# speedup vs baseline: 1.1851x; 1.1135x over previous
import functools

import jax
import jax.numpy as jnp
from jax import lax
from jax.experimental import pallas as pl
from jax.experimental.pallas import tpu as pltpu
from jax.experimental.pallas import tpu_sc as plsc

B, L = 8, 512
HID, NFD, EFD = 256, 64, 32
K, NL = 30, 3
CUTOFF = 20.0
NN = B * L
E = NN * K

F32 = jnp.float32


def _comps(v):
    return v[:, 0:1], v[:, 1:2], v[:, 2:3]


def _dihedral_sincos(p1, p2, p3, p4):
    v1 = p2 - p1
    v2 = p3 - p2
    v3 = p4 - p3
    v1x, v1y, v1z = _comps(v1)
    v2x, v2y, v2z = _comps(v2)
    v3x, v3y, v3z = _comps(v3)
    n1x = v1y * v2z - v1z * v2y
    n1y = v1z * v2x - v1x * v2z
    n1z = v1x * v2y - v1y * v2x
    n2x = v2y * v3z - v2z * v3y
    n2y = v2z * v3x - v2x * v3z
    n2z = v2x * v3y - v2y * v3x
    n1n = jnp.sqrt(n1x * n1x + n1y * n1y + n1z * n1z) + 1e-8
    n2n = jnp.sqrt(n2x * n2x + n2y * n2y + n2z * n2z) + 1e-8
    u1x, u1y, u1z = n1x / n1n, n1y / n1n, n1z / n1n
    u2x, u2y, u2z = n2x / n2n, n2y / n2n, n2z / n2n
    c = jnp.clip(u1x * u2x + u1y * u2y + u1z * u2z, -1.0, 1.0)
    sgn = jnp.sign(u1x * v3x + u1y * v3y + u1z * v3z)
    s = sgn * jnp.sqrt(jnp.maximum((1.0 - c) * (1.0 + c), 0.0))
    c = jnp.where(sgn == 0.0, 1.0, c)
    return s, c


def _ln(x, g, b):
    mu = jnp.mean(x, axis=-1, keepdims=True)
    var = jnp.mean((x - mu) ** 2, axis=-1, keepdims=True)
    return (x - mu) / jnp.sqrt(var + 1e-5) * g + b


def _dot(a, b):
    return jnp.dot(a, b, preferred_element_type=F32)


_GW = HID // 2


def _pack_bf16(y):
    ua = lax.bitcast_convert_type(y[:, :_GW], jnp.uint32)
    ub = lax.bitcast_convert_type(y[:, _GW:], jnp.uint32)
    ra = (ua + jnp.uint32(0x7FFF) + ((ua >> 16) & jnp.uint32(1))) >> 16
    rb = (ub + jnp.uint32(0x7FFF) + ((ub >> 16) & jnp.uint32(1))) >> 16
    return lax.bitcast_convert_type(ra | (rb << 16), F32)


def _unpack_bf16(w):
    u = lax.bitcast_convert_type(w, jnp.uint32)
    a = lax.bitcast_convert_type(u << 16, F32)
    b = lax.bitcast_convert_type(u & jnp.uint32(0xFFFF0000), F32)
    return jnp.concatenate([a, b], axis=1)


_BPP = 1


def _enc_knn_body(cprev_r, n_r, ca_r, c_r, nnext_r, nnom_r, canom_r, cat_r,
                  wd_r, bd_r, wn_r, bn_r, w1a_r, b1_r,
                  x_o, y_o, nbr_o, dist_o):
    for bb in range(_BPP):
        bidx = pl.program_id(0) * _BPP + bb
        n_ = n_r[bb]
        ca = ca_r[bb]
        c = c_r[bb]
        s1, c1 = _dihedral_sincos(cprev_r[bb], n_, ca, c)
        s2, c2 = _dihedral_sincos(n_, ca, c, nnext_r[bb])
        s3, c3 = _dihedral_sincos(ca, c, nnom_r[bb], canom_r[bb])
        enc = jnp.concatenate([s1, s2, s3, c1, c2, c3], axis=1)
        nf = _dot(enc, wd_r[...]) + bd_r[...]
        x = _dot(nf, wn_r[...]) + bn_r[...]
        x_o[bb * L:(bb + 1) * L, :] = x
        y_o[bb * L:(bb + 1) * L, :] = _pack_bf16(_dot(x, w1a_r[...]) + b1_r[...])

        cat = cat_r[bb]
        dx = ca[:, 0:1] - cat[0:1, :]
        dy = ca[:, 1:2] - cat[1:2, :]
        dz = ca[:, 2:3] - cat[2:3, :]
        ri = lax.broadcasted_iota(jnp.int32, (L, L), 0)
        ci = lax.broadcasted_iota(jnp.int32, (L, L), 1)
        d2 = dx * dx + dy * dy + dz * dz + jnp.where(ri == ci, F32(1e9), F32(0.0))
        work = -d2
        nbrs = []
        dists = []
        for _ in range(K):
            mx = jnp.max(work, axis=1, keepdims=True)
            sel = jnp.where(work == mx, ci, jnp.int32(1 << 30))
            idx = jnp.min(sel, axis=1, keepdims=True)
            nbrs.append(idx)
            dists.append(jnp.sqrt(jnp.maximum(-mx, 0.0) + 1e-12))
            work = jnp.where(ci == idx, F32(-3e38), work)
        nbr_o[bb] = jnp.concatenate(nbrs, axis=1) + bidx * L
        dist_o[bb] = jnp.concatenate(dists, axis=1)


_COORD_SPEC = pl.BlockSpec((_BPP, L, 3), lambda i: (i, 0, 0))


def _wspec(shape):
    return pl.BlockSpec(shape, lambda i: tuple(0 for _ in shape))


_enc_knn_call = pl.pallas_call(
    _enc_knn_body,
    grid=(B // _BPP,),
    in_specs=[_COORD_SPEC] * 7 + [pl.BlockSpec((_BPP, 3, L), lambda i: (i, 0, 0)),
                                  _wspec((6, NFD)), _wspec((1, NFD)),
                                  _wspec((NFD, HID)), _wspec((1, HID)),
                                  _wspec((HID, HID)), _wspec((1, HID))],
    out_specs=[pl.BlockSpec((_BPP * L, HID), lambda i: (i, 0)),
               pl.BlockSpec((_BPP * L, HID // 2), lambda i: (i, 0)),
               pl.BlockSpec((_BPP, L, K), lambda i: (i, 0, 0)),
               pl.BlockSpec((_BPP, L, K), lambda i: (i, 0, 0))],
    out_shape=[jax.ShapeDtypeStruct((NN, HID), F32),
               jax.ShapeDtypeStruct((NN, HID // 2), F32),
               jax.ShapeDtypeStruct((B, L, K), jnp.int32),
               jax.ShapeDtypeStruct((B, L, K), F32)],
)


_NW = 32
_EPW = E // _NW
_CH = 128
_NCH = _EPW // _CH


def _sc_gather_body(y_hbm, src_hbm, out_hbm, idx_v, rows_v, sem):
    wid = lax.axis_index("s") * 2 + lax.axis_index("c")
    base = pl.multiple_of(wid * _EPW, _CH)
    pltpu.sync_copy(src_hbm.at[pl.ds(base, _EPW)], idx_v)

    def chunk(j, carry):
        off = pl.multiple_of(base + j * _CH, _CH)
        pltpu.async_copy(y_hbm.at[idx_v.at[pl.ds(j * _CH, _CH)]], rows_v,
                         sem).wait()
        pltpu.sync_copy(rows_v, out_hbm.at[pl.ds(off, _CH)])
        return carry

    lax.fori_loop(0, _NCH, chunk, 0)


@functools.lru_cache(maxsize=1)
def _sc_gather_call():
    return pl.kernel(
        _sc_gather_body,
        out_type=jax.ShapeDtypeStruct((E, _GW), F32),
        mesh=plsc.VectorSubcoreMesh(core_axis_name="c", subcore_axis_name="s"),
        scratch_types=[pltpu.VMEM((_EPW,), jnp.int32),
                       pltpu.VMEM((_CH, _GW), F32),
                       pltpu.SemaphoreType.DMA],
    )


_NT = 256
_TE = _NT * K
_NTILES = NN // _NT


def _layer_body(g_r, d_r, x_r,
                w1b_r, mg1_r, mB1_r, mW2_r, mb2_r,
                uW1a_r, uW1b_r, ub1_r, ug1_r, uB1_r, uW2_r, ub2_r,
                w1an_r, b1n_r,
                x_o, y_o):
    dist = d_r[...]
    ii = lax.broadcasted_iota(jnp.int32, (1, EFD), 1).astype(F32)
    cen = ii * (CUTOFF / (EFD - 1))
    z = (dist - cen) * (EFD / CUTOFF)
    ea = jnp.exp(-0.5 * z * z)
    m = _unpack_bf16(g_r[...]) + _dot(ea, w1b_r[...])
    m = jnp.maximum(_ln(m, mg1_r[...], mB1_r[...]), 0.0)
    s = jnp.sum(m.reshape(_NT, K, HID), axis=1)
    agg = _dot(s, mW2_r[...]) + F32(K) * mb2_r[...]
    u = _dot(x_r[...], uW1a_r[...]) + _dot(agg, uW1b_r[...]) + ub1_r[...]
    u = jnp.maximum(_ln(u, ug1_r[...], uB1_r[...]), 0.0)
    xn = _dot(u, uW2_r[...]) + ub2_r[...]
    x_o[...] = xn
    y_o[...] = _pack_bf16(_dot(xn, w1an_r[...]) + b1n_r[...])


_layer_call = pl.pallas_call(
    _layer_body,
    grid=(_NTILES,),
    in_specs=[pl.BlockSpec((_TE, HID // 2), lambda i: (i, 0)),
              pl.BlockSpec((_TE, 1), lambda i: (i, 0)),
              pl.BlockSpec((_NT, HID), lambda i: (i, 0)),
              _wspec((EFD, HID)), _wspec((1, HID)), _wspec((1, HID)),
              _wspec((HID, HID)), _wspec((1, HID)),
              _wspec((HID, HID)), _wspec((HID, HID)), _wspec((1, HID)),
              _wspec((1, HID)), _wspec((1, HID)), _wspec((HID, HID)),
              _wspec((1, HID)),
              _wspec((HID, HID)), _wspec((1, HID))],
    out_specs=[pl.BlockSpec((_NT, HID), lambda i: (i, 0)),
               pl.BlockSpec((_NT, HID // 2), lambda i: (i, 0))],
    out_shape=[jax.ShapeDtypeStruct((NN, HID), F32),
               jax.ShapeDtypeStruct((NN, HID // 2), F32)],
)


def _row(v):
    return v.reshape(1, -1)


def kernel(backbone_coords, params):
    coords = backbone_coords
    n_ = coords[:, :, 0, :]
    ca = coords[:, :, 1, :]
    c = coords[:, :, 2, :]
    cprev = jnp.concatenate([c[:, :1], c[:, :-1]], axis=1)
    nnext = jnp.concatenate([n_[:, 1:], n_[:, -1:]], axis=1)
    nnom = jnp.concatenate([n_[:, 1:], ca[:, -1:]], axis=1)
    canom = jnp.concatenate([ca[:, 1:], ca[:, -1:]], axis=1)
    cat = ca.transpose(0, 2, 1)

    lp0 = params["layers"][0]
    x, y, nbrg, dist = _enc_knn_call(
        cprev, n_, ca, c, nnext, nnom, canom, cat,
        params["dproj"]["W"], _row(params["dproj"]["b"]),
        params["nproj"]["W"], _row(params["nproj"]["b"]),
        lp0["mW1"][:HID], _row(lp0["mb1"]))

    src = nbrg.reshape(E)
    dist_e = dist.reshape(E, 1)
    zW = jnp.zeros((HID, HID), F32)
    zb = jnp.zeros((1, HID), F32)

    for l in range(NL):
        lp = params["layers"][l]
        if l + 1 < NL:
            lpn = params["layers"][l + 1]
            w1an, b1n = lpn["mW1"][:HID], _row(lpn["mb1"])
        else:
            w1an, b1n = zW, zb
        g = _sc_gather_call()(y, src)
        x, y = _layer_call(
            g, dist_e, x,
            lp["mW1"][HID:], _row(lp["mg1"]), _row(lp["mB1"]),
            lp["mW2"], _row(lp["mb2"]),
            lp["uW1"][:HID], lp["uW1"][HID:], _row(lp["ub1"]),
            _row(lp["ug1"]), _row(lp["uB1"]), lp["uW2"], _row(lp["ub2"]),
            w1an, b1n)

    return x.reshape(B, L, HID)
